# Initial kernel scaffold; baseline (speedup 1.0000x reference)
import jax
import jax.numpy as jnp
from jax.experimental import pallas as pl


def kernel(features, edge_index, W1, b1, W2, b2, W3, b3):
    raise NotImplementedError("write your pallas kernel here")


# speedup vs baseline: 3.3102x; 3.3102x over previous
import functools

import jax
import jax.numpy as jnp
from jax import lax
from jax.experimental import pallas as pl
from jax.experimental.pallas import tpu as pltpu
from jax.experimental.pallas import tpu_sc as plsc

N = 10000
E = 160000
D = 256
DH = D // 2
NC = 2
NS = 16
LANES = 16
DEGW = 128

EPT = E // NS
EB = 80
NB = EPT // EB

ACC_ROWS = 10240
ZROWS = ACC_ROWS // NS
OBLKS = N // EB

MBLK = 400
GRID = N // MBLK

_mesh = plsc.VectorSubcoreMesh(core_axis_name="c", subcore_axis_name="s")


def _copy_out(acc_sh, out_hbm, s):
  nblk = jnp.where(s < OBLKS % NS, OBLKS // NS + 1, OBLKS // NS)
  def body(t, _):
    row = (s + t * NS) * EB
    pltpu.sync_copy(acc_sh.at[pl.ds(row, EB)], out_hbm.at[pl.ds(row, EB)])
    return 0
  lax.fori_loop(0, nblk, body, 0)


def _fill_2d(ref, rows, width, value):
  def body(r, _):
    for j in range(width // LANES):
      ref[r, pl.ds(j * LANES, LANES)] = jnp.full((LANES,), value, jnp.float32)
    return 0
  lax.fori_loop(0, rows, body, 0)


@functools.partial(
    pl.kernel,
    out_type=(jax.ShapeDtypeStruct((N, DEGW), jnp.float32),
              jax.ShapeDtypeStruct((N, DEGW), jnp.float32)),
    mesh=_mesh,
    scratch_types=[
        pltpu.VMEM((EB,), jnp.int32),
        pltpu.VMEM((EB, DEGW), jnp.float32),
        pltpu.VMEM_SHARED((ACC_ROWS, DEGW), jnp.float32),
    ],
)
def _deg_kernel(src_hbm, dst_hbm, degout_hbm, degin_hbm, idx_v, pay_v, acc_sh):
  c = lax.axis_index("c")
  s = lax.axis_index("s")

  _fill_2d(pay_v, EB, DEGW, 0.0)
  for k in range(ZROWS // EB):
    pltpu.sync_copy(pay_v, acc_sh.at[pl.ds(s * ZROWS + k * EB, EB)])
  plsc.subcore_barrier()

  _fill_2d(pay_v, EB, DEGW, 1.0)

  def run(idx_hbm, out_hbm):
    def body(i, _):
      base = s * EPT + i * EB
      pltpu.sync_copy(idx_hbm.at[pl.ds(base, EB)], idx_v)
      pltpu.sync_copy(pay_v, acc_sh.at[idx_v], add=True)
      return 0
    lax.fori_loop(0, NB, body, 0)
    plsc.subcore_barrier()
    _copy_out(acc_sh, out_hbm, s)

  @pl.when(c == 0)
  def _():
    run(src_hbm, degout_hbm)

  @pl.when(c == 1)
  def _():
    run(dst_hbm, degin_hbm)


@functools.partial(
    pl.kernel,
    out_type=(jax.ShapeDtypeStruct((N, DH), jnp.float32),
              jax.ShapeDtypeStruct((N, DH), jnp.float32)),
    mesh=_mesh,
    scratch_types=[
        pltpu.VMEM((EB,), jnp.int32),
        pltpu.VMEM((EB,), jnp.int32),
        pltpu.VMEM((EB, DH), jnp.float32),
        pltpu.VMEM_SHARED((ACC_ROWS, DH), jnp.float32),
        pltpu.SemaphoreType.DMA,
    ],
)
def _agg_kernel(hl_hbm, hr_hbm, src_hbm, dst_hbm, outl_hbm, outr_hbm,
                srcv, dstv, msgs, acc_sh, sem):
  c = lax.axis_index("c")
  s = lax.axis_index("s")

  _fill_2d(msgs, EB, DH, 0.0)
  for k in range(ZROWS // EB):
    pltpu.sync_copy(msgs, acc_sh.at[pl.ds(s * ZROWS + k * EB, EB)])
  plsc.subcore_barrier()

  def run(h_hbm, out_hbm):
    def body(i, _):
      base = s * EPT + i * EB
      pltpu.sync_copy(src_hbm.at[pl.ds(base, EB)], srcv)
      pltpu.sync_copy(dst_hbm.at[pl.ds(base, EB)], dstv)
      pltpu.async_copy(h_hbm.at[srcv], msgs, sem).wait()
      pltpu.sync_copy(msgs, acc_sh.at[dstv], add=True)
      return 0
    lax.fori_loop(0, NB, body, 0)
    plsc.subcore_barrier()
    _copy_out(acc_sh, out_hbm, s)

  @pl.when(c == 0)
  def _():
    run(hl_hbm, outl_hbm)

  @pl.when(c == 1)
  def _():
    run(hr_hbm, outr_hbm)


def _prep_body(feat_ref, dow_ref, diw_ref, hl_ref, hr_ref, nsw_ref, ndw_ref):
  ns = lax.rsqrt(jnp.maximum(dow_ref[...], 1.0))
  nd = lax.rsqrt(jnp.maximum(diw_ref[...], 1.0))
  nsw_ref[...] = ns
  ndw_ref[...] = nd
  h0s = feat_ref[...] * ns[:, 0:1]
  hl_ref[...] = h0s[:, :DH]
  hr_ref[...] = h0s[:, DH:]


def _prep(features, degout_w, degin_w):
  return pl.pallas_call(
      _prep_body,
      grid=(GRID,),
      in_specs=[
          pl.BlockSpec((MBLK, D), lambda i: (i, 0)),
          pl.BlockSpec((MBLK, DEGW), lambda i: (i, 0)),
          pl.BlockSpec((MBLK, DEGW), lambda i: (i, 0)),
      ],
      out_specs=[
          pl.BlockSpec((MBLK, DH), lambda i: (i, 0)),
          pl.BlockSpec((MBLK, DH), lambda i: (i, 0)),
          pl.BlockSpec((MBLK, DEGW), lambda i: (i, 0)),
          pl.BlockSpec((MBLK, DEGW), lambda i: (i, 0)),
      ],
      out_shape=[
          jax.ShapeDtypeStruct((N, DH), jnp.float32),
          jax.ShapeDtypeStruct((N, DH), jnp.float32),
          jax.ShapeDtypeStruct((N, DEGW), jnp.float32),
          jax.ShapeDtypeStruct((N, DEGW), jnp.float32),
      ],
  )(features, degout_w, degin_w)


def _layer_body_mid(al_ref, ar_ref, ndw_ref, nsw_ref, w_ref, b_ref,
                    outl_ref, outr_ref):
  nd = ndw_ref[:, 0:1]
  y = (jnp.dot(al_ref[...] * nd, w_ref[:DH, :],
               preferred_element_type=jnp.float32)
       + jnp.dot(ar_ref[...] * nd, w_ref[DH:, :],
                 preferred_element_type=jnp.float32)
       + b_ref[0:1, :])
  y = jnp.maximum(y, 0.0) * nsw_ref[:, 0:1]
  outl_ref[...] = y[:, :DH]
  outr_ref[...] = y[:, DH:]


def _layer_body_last(al_ref, ar_ref, ndw_ref, nsw_ref, w_ref, b_ref, out_ref):
  nd = ndw_ref[:, 0:1]
  out_ref[...] = (jnp.dot(al_ref[...] * nd, w_ref[:DH, :],
                          preferred_element_type=jnp.float32)
                  + jnp.dot(ar_ref[...] * nd, w_ref[DH:, :],
                            preferred_element_type=jnp.float32)
                  + b_ref[0:1, :])


def _layer(al, ar, ndw, nsw, w, b, last):
  if last:
    out_specs = pl.BlockSpec((MBLK, D), lambda i: (i, 0))
    out_shape = jax.ShapeDtypeStruct((N, D), jnp.float32)
    body = _layer_body_last
  else:
    out_specs = [pl.BlockSpec((MBLK, DH), lambda i: (i, 0)),
                 pl.BlockSpec((MBLK, DH), lambda i: (i, 0))]
    out_shape = [jax.ShapeDtypeStruct((N, DH), jnp.float32),
                 jax.ShapeDtypeStruct((N, DH), jnp.float32)]
    body = _layer_body_mid
  return pl.pallas_call(
      body,
      grid=(GRID,),
      in_specs=[
          pl.BlockSpec((MBLK, DH), lambda i: (i, 0)),
          pl.BlockSpec((MBLK, DH), lambda i: (i, 0)),
          pl.BlockSpec((MBLK, DEGW), lambda i: (i, 0)),
          pl.BlockSpec((MBLK, DEGW), lambda i: (i, 0)),
          pl.BlockSpec((D, D), lambda i: (0, 0)),
          pl.BlockSpec((1, D), lambda i: (0, 0)),
      ],
      out_specs=out_specs,
      out_shape=out_shape,
  )(al, ar, ndw, nsw, w, b.reshape(1, D))


def kernel(features, edge_index, W1, b1, W2, b2, W3, b3):
  src = edge_index[0].astype(jnp.int32)
  dst = edge_index[1].astype(jnp.int32)

  degout_w, degin_w = _deg_kernel(src, dst)
  hl, hr, nsw, ndw = _prep(features, degout_w, degin_w)

  al, ar = _agg_kernel(hl, hr, src, dst)
  hl, hr = _layer(al, ar, ndw, nsw, W1, b1, last=False)

  al, ar = _agg_kernel(hl, hr, src, dst)
  hl, hr = _layer(al, ar, ndw, nsw, W2, b2, last=False)

  al, ar = _agg_kernel(hl, hr, src, dst)
  return _layer(al, ar, ndw, nsw, W3, b3, last=True)
